# Initial kernel scaffold; baseline (speedup 1.0000x reference)
import jax
import jax.numpy as jnp
from jax.experimental import pallas as pl


def kernel(samples, contexts, gamma, beta, moving_mean, moving_var, priors):
    raise NotImplementedError("write your pallas kernel here")


# speedup vs baseline: 8.8765x; 8.8765x over previous
import functools

import jax
import jax.numpy as jnp
from jax.experimental import pallas as pl

EPS = 0.001


def _fold_params_kernel(g_ref, b_ref, m_ref, v_ref, p_ref, a_out, b_out):
    inv = jax.lax.rsqrt(v_ref[...] + EPS) * g_ref[...]
    rp = jax.lax.rsqrt(p_ref[...])
    a_out[...] = inv * rp
    b_out[...] = (b_ref[...] - m_ref[...] * inv) * rp


def _apply_kernel(x_ref, c_ref, a_ref, b_ref, o_ref):
    c = c_ref[...]
    onehot = (c == jax.lax.broadcasted_iota(jnp.int32, (c.shape[0], a_ref.shape[0]), 1)
              ).astype(jnp.float32)
    a_sel = jnp.dot(onehot, a_ref[...], preferred_element_type=jnp.float32)
    b_sel = jnp.dot(onehot, b_ref[...], preferred_element_type=jnp.float32)
    o_ref[...] = x_ref[...] * a_sel + b_sel


def kernel(samples, contexts, gamma, beta, moving_mean, moving_var, priors):
    N, D = samples.shape
    C = gamma.shape[0]
    a_tab, b_tab = pl.pallas_call(
        _fold_params_kernel,
        out_shape=(jax.ShapeDtypeStruct((C, D), jnp.float32),
                   jax.ShapeDtypeStruct((C, D), jnp.float32)),
    )(gamma, beta, moving_mean, moving_var, priors.reshape(C, 1))

    BN = 2048
    grid = (N // BN,)
    out = pl.pallas_call(
        _apply_kernel,
        grid=grid,
        in_specs=[
            pl.BlockSpec((BN, D), lambda i: (i, 0)),
            pl.BlockSpec((BN, 1), lambda i: (i, 0)),
            pl.BlockSpec((C, D), lambda i: (0, 0)),
            pl.BlockSpec((C, D), lambda i: (0, 0)),
        ],
        out_specs=pl.BlockSpec((BN, D), lambda i: (i, 0)),
        out_shape=jax.ShapeDtypeStruct((N, D), jnp.float32),
    )(samples, contexts, a_tab, b_tab)
    return out
